# baseline (speedup 1.0000x reference)
import functools

import jax
import jax.numpy as jnp
from jax import lax
from jax.experimental import pallas as pl
from jax.experimental.pallas import tpu as pltpu
from jax.experimental.pallas import tpu_sc as plsc

_NUM_DOMAINS = 1000
_PLEN = 10
_EDIM = 768
_BATCH = 4096
_NC, _NS = 2, 16
_NW = _NC * _NS
_BPW = _BATCH // _NW
_CH = 16
_CPP = _BPW // _CH
_NBUF = 8
_LANES = 16

_mesh = plsc.VectorSubcoreMesh(core_axis_name="c", subcore_axis_name="s")


@functools.partial(
    pl.kernel,
    mesh=_mesh,
    out_type=jax.ShapeDtypeStruct((_PLEN * _BATCH, _EDIM), jnp.float32),
    scratch_types=[
        pltpu.VMEM((_BPW,), jnp.int32),
        pltpu.VMEM((_PLEN * _BPW,), jnp.int32),
        pltpu.VMEM((_NBUF, _CH, _EDIM), jnp.float32),
        pltpu.SemaphoreType.DMA,
        pltpu.SemaphoreType.DMA,
    ],
)
def _gather_rows(idx_hbm, table_hbm, out_hbm, idx_v, idxs_v, rows_v, gsem, ssem):
    wid = lax.axis_index("s") * _NC + lax.axis_index("c")
    base = wid * _BPW
    pltpu.sync_copy(idx_hbm.at[pl.ds(base, _BPW)], idx_v)

    chunks = [(p, cc) for p in range(_PLEN) for cc in range(_CPP)]
    n = len(chunks)

    def gather(k, buf):
        p, cc = chunks[k]
        if p == 0:
            idx_ref = idx_v.at[pl.ds(cc * _CH, _CH)]
        else:
            idx_ref = idxs_v.at[pl.ds(p * _BPW + cc * _CH, _CH)]
        return pltpu.async_copy(table_hbm.at[idx_ref], rows_v.at[buf], gsem)

    def put(k, buf):
        p, cc = chunks[k]
        return pltpu.async_copy(
            rows_v.at[buf],
            out_hbm.at[pl.ds(p * _BATCH + base + cc * _CH, _CH)],
            ssem,
        )

    g = [None] * _NBUF
    s = [None] * _NBUF
    for i in range(_NBUF - 1):
        g[i] = gather(i, i)

    for p in range(1, _PLEN):
        for j in range(0, _BPW, _LANES):
            idxs_v[pl.ds(p * _BPW + j, _LANES)] = (
                idx_v[pl.ds(j, _LANES)] + p * _NUM_DOMAINS
            )

    for k in range(n):
        buf = k % _NBUF
        g[buf].wait()
        s[buf] = put(k, buf)
        nk = k + _NBUF - 1
        if nk < n:
            nbuf = nk % _NBUF
            if s[nbuf] is not None:
                s[nbuf].wait()
            g[nbuf] = gather(nk, nbuf)
    for i in range(_NBUF):
        if s[i] is not None:
            s[i].wait()


def kernel(domain_labels, domain_prompts):
    idx = domain_labels.astype(jnp.int32)
    tab2 = jnp.transpose(domain_prompts, (1, 0, 2)).reshape(
        _PLEN * _NUM_DOMAINS, _EDIM
    )
    out2 = _gather_rows(idx, tab2)
    out = jnp.transpose(out2.reshape(_PLEN, _BATCH, _EDIM), (1, 0, 2))
    return out


# speedup vs baseline: 1.0071x; 1.0071x over previous
import functools

import jax
import jax.numpy as jnp
from jax import lax
from jax.experimental import pallas as pl
from jax.experimental.pallas import tpu as pltpu
from jax.experimental.pallas import tpu_sc as plsc

_NUM_DOMAINS = 1000
_PLEN = 10
_EDIM = 768
_BATCH = 4096
_NC, _NS = 2, 16
_NW = _NC * _NS
_BPW = _BATCH // _NW
_CH = 32
_CPP = _BPW // _CH
_NBUF = 5
_LANES = 16

_mesh = plsc.VectorSubcoreMesh(core_axis_name="c", subcore_axis_name="s")


@functools.partial(
    pl.kernel,
    mesh=_mesh,
    out_type=jax.ShapeDtypeStruct((_PLEN * _BATCH, _EDIM), jnp.float32),
    scratch_types=[
        pltpu.VMEM((_BPW,), jnp.int32),
        pltpu.VMEM((_PLEN * _BPW,), jnp.int32),
        pltpu.VMEM((_NBUF, _CH, _EDIM), jnp.float32),
        pltpu.SemaphoreType.DMA,
        pltpu.SemaphoreType.DMA,
    ],
)
def _gather_rows(idx_hbm, table_hbm, out_hbm, idx_v, idxs_v, rows_v, gsem, ssem):
    wid = lax.axis_index("s") * _NC + lax.axis_index("c")
    base = wid * _BPW
    pltpu.sync_copy(idx_hbm.at[pl.ds(base, _BPW)], idx_v)

    chunks = [(p, cc) for p in range(_PLEN) for cc in range(_CPP)]
    n = len(chunks)

    def gather(k, buf):
        p, cc = chunks[k]
        if p == 0:
            idx_ref = idx_v.at[pl.ds(cc * _CH, _CH)]
        else:
            idx_ref = idxs_v.at[pl.ds(p * _BPW + cc * _CH, _CH)]
        return pltpu.async_copy(table_hbm.at[idx_ref], rows_v.at[buf], gsem)

    def put(k, buf):
        p, cc = chunks[k]
        return pltpu.async_copy(
            rows_v.at[buf],
            out_hbm.at[pl.ds(p * _BATCH + base + cc * _CH, _CH)],
            ssem,
        )

    g = [None] * _NBUF
    s = [None] * _NBUF
    for i in range(_NBUF - 1):
        g[i] = gather(i, i)

    for p in range(1, _PLEN):
        for j in range(0, _BPW, _LANES):
            idxs_v[pl.ds(p * _BPW + j, _LANES)] = (
                idx_v[pl.ds(j, _LANES)] + p * _NUM_DOMAINS
            )

    for k in range(n):
        buf = k % _NBUF
        g[buf].wait()
        s[buf] = put(k, buf)
        nk = k + _NBUF - 1
        if nk < n:
            nbuf = nk % _NBUF
            if s[nbuf] is not None:
                s[nbuf].wait()
            g[nbuf] = gather(nk, nbuf)
    for i in range(_NBUF):
        if s[i] is not None:
            s[i].wait()


def kernel(domain_labels, domain_prompts):
    idx = domain_labels.astype(jnp.int32)
    tab2 = jnp.transpose(domain_prompts, (1, 0, 2)).reshape(
        _PLEN * _NUM_DOMAINS, _EDIM
    )
    out2 = _gather_rows(idx, tab2)
    out = jnp.transpose(out2.reshape(_PLEN, _BATCH, _EDIM), (1, 0, 2))
    return out
